# Initial kernel scaffold; baseline (speedup 1.0000x reference)
import jax
import jax.numpy as jnp
from jax.experimental import pallas as pl


def kernel(basic_block, edge_index, W1, b1, W2, b2):
    raise NotImplementedError("write your pallas kernel here")


# speedup vs baseline: 8.6581x; 8.6581x over previous
import functools

import jax
import jax.numpy as jnp
from jax import lax
from jax.experimental import pallas as pl
from jax.experimental.pallas import tpu as pltpu
from jax.experimental.pallas import tpu_sc as plsc

N_NODES = 10000
D = 128
N_EDGES = 320000

NC, NS = 2, 16
NW = NC * NS
CHUNK = 128
K = 80
EP = NW * K * CHUNK
NP = 10240
RPT = NP // NS
DW = 16

_mesh = plsc.VectorSubcoreMesh(
    core_axis_name="c", subcore_axis_name="s", num_cores=NC, num_subcores=NS
)


_deg_wrap = functools.partial(
    pl.kernel,
    out_type=jax.ShapeDtypeStruct((NC, NP), jnp.float32),
    mesh=_mesh,
    scratch_types=[
        pltpu.VMEM((K, CHUNK), jnp.int32),
        pltpu.VMEM((CHUNK,), jnp.float32),
        pltpu.VMEM((RPT,), jnp.float32),
        pltpu.VMEM_SHARED((NP,), jnp.float32),
    ],
)


def _deg_body(dst_hbm, out_hbm, idx_v, ones_v, zeros_v, acc):
    c = lax.axis_index("c")
    s = lax.axis_index("s")
    wid = c * NS + s

    def fill_z(r, _):
        zeros_v[pl.ds(r * 16, 16)] = jnp.zeros((16,), jnp.float32)
        return 0

    lax.fori_loop(0, RPT // 16, fill_z, 0)

    def fill_o(r, _):
        ones_v[pl.ds(r * 16, 16)] = jnp.full((16,), 1.0, jnp.float32)
        return 0

    lax.fori_loop(0, CHUNK // 16, fill_o, 0)

    pltpu.sync_copy(zeros_v, acc.at[pl.ds(s * RPT, RPT)])
    pltpu.sync_copy(dst_hbm.at[wid], idx_v)
    plsc.subcore_barrier()

    def body(j, _):
        pltpu.sync_copy(ones_v, acc.at[idx_v.at[j]], add=True)
        return 0

    lax.fori_loop(0, K, body, 0)
    plsc.subcore_barrier()
    pltpu.sync_copy(acc.at[pl.ds(s * RPT, RPT)], out_hbm.at[c, pl.ds(s * RPT, RPT)])


_msg_wrap = functools.partial(
    pl.kernel,
    out_type=jax.ShapeDtypeStruct((NC, NP, D), jnp.float32),
    mesh=_mesh,
    scratch_types=[
        pltpu.VMEM((K, CHUNK), jnp.int32),
        pltpu.VMEM((K, CHUNK), jnp.int32),
        pltpu.VMEM((CHUNK, D), jnp.float32),
        pltpu.VMEM_SHARED((NP, D), jnp.float32),
        pltpu.SemaphoreType.DMA,
    ],
)


def _msg_body(y_hbm, src_hbm, dst_hbm, out_hbm, src_v, dst_v, buf0, acc, sem0):
    c = lax.axis_index("c")
    s = lax.axis_index("s")
    wid = c * NS + s

    def fill(r, _):
        for q in range(D // 16):
            buf0[r, pl.ds(q * 16, 16)] = jnp.zeros((16,), jnp.float32)
        return 0

    lax.fori_loop(0, CHUNK, fill, 0)

    def zblk(b, _):
        pltpu.sync_copy(buf0, acc.at[pl.ds(s * RPT + b * CHUNK, CHUNK)])
        return 0

    lax.fori_loop(0, RPT // CHUNK, zblk, 0)
    pltpu.sync_copy(src_hbm.at[wid], src_v)
    pltpu.sync_copy(dst_hbm.at[wid], dst_v)
    plsc.subcore_barrier()

    def body(j, _):
        pltpu.async_copy(y_hbm.at[src_v.at[j]], buf0, sem0).wait()
        pltpu.sync_copy(buf0, acc.at[dst_v.at[j]], add=True)
        return 0

    lax.fori_loop(0, K, body, 0)
    plsc.subcore_barrier()
    pltpu.sync_copy(acc.at[pl.ds(s * RPT, RPT)], out_hbm.at[c, pl.ds(s * RPT, RPT)])


_deg_kernel = _deg_wrap(_deg_body)
_msg_kernel = _msg_wrap(_msg_body)


def _tc_pre_body(x_ref, w_ref, deg_ref, xw_ref, y_ref):
    xw = jnp.dot(x_ref[...], w_ref[...], preferred_element_type=jnp.float32)
    deg = deg_ref[0, 0:N_NODES, :] + deg_ref[1, 0:N_NODES, :] + 1.0
    dinv = lax.rsqrt(deg)
    xw_ref[...] = xw
    y_ref[...] = xw * dinv


def _tc_pre(x, W1, degp):
    return pl.pallas_call(
        _tc_pre_body,
        out_shape=(
            jax.ShapeDtypeStruct((N_NODES, D), jnp.float32),
            jax.ShapeDtypeStruct((N_NODES, D), jnp.float32),
        ),
    )(x, W1, degp)


def _tc_mid_body(z_ref, deg_ref, xw1_ref, w2_ref, b1_ref, xw2_ref, y2_ref):
    z = z_ref[0, 0:N_NODES, :] + z_ref[1, 0:N_NODES, :]
    deg = deg_ref[0, 0:N_NODES, :] + deg_ref[1, 0:N_NODES, :] + 1.0
    dinv = lax.rsqrt(deg)
    h = jnp.maximum(z * dinv + xw1_ref[...] * (dinv * dinv) + b1_ref[...], 0.0)
    xw2 = jnp.dot(h, w2_ref[...], preferred_element_type=jnp.float32)
    xw2_ref[...] = xw2
    y2_ref[...] = xw2 * dinv


def _tc_mid(z1p, degp, xw1, W2, b1):
    return pl.pallas_call(
        _tc_mid_body,
        out_shape=(
            jax.ShapeDtypeStruct((N_NODES, D), jnp.float32),
            jax.ShapeDtypeStruct((N_NODES, D), jnp.float32),
        ),
    )(z1p, degp, xw1, W2, b1)


def _tc_post_body(z_ref, deg_ref, xw2_ref, b2_ref, out_ref):
    z = z_ref[0, 0:N_NODES, :] + z_ref[1, 0:N_NODES, :]
    deg = deg_ref[0, 0:N_NODES, :] + deg_ref[1, 0:N_NODES, :] + 1.0
    dinv = lax.rsqrt(deg)
    out_ref[...] = z * dinv + xw2_ref[...] * (dinv * dinv) + b2_ref[...]


def _tc_post(z2p, degp, xw2, b2):
    return pl.pallas_call(
        _tc_post_body,
        out_shape=jax.ShapeDtypeStruct((N_NODES, D), jnp.float32),
    )(z2p, degp, xw2, b2)


def kernel(basic_block, edge_index, W1, b1, W2, b2):
    src = edge_index[0].astype(jnp.int32)
    dst = edge_index[1].astype(jnp.int32)
    pad = EP - N_EDGES
    src_p = jnp.concatenate([src, jnp.zeros((pad,), jnp.int32)]).reshape(NW, K, CHUNK)
    dst_p = jnp.concatenate([dst, jnp.full((pad,), NP - 1, jnp.int32)]).reshape(NW, K, CHUNK)

    degp = _deg_kernel(dst_p).reshape(NC, NP, 1)
    xw1, y1 = _tc_pre(basic_block, W1, degp)
    z1p = _msg_kernel(y1, src_p, dst_p)
    xw2, y2 = _tc_mid(z1p, degp, xw1, W2, b1.reshape(1, D))
    z2p = _msg_kernel(y2, src_p, dst_p)
    return _tc_post(z2p, degp, xw2, b2.reshape(1, D))


# speedup vs baseline: 9.7166x; 1.1223x over previous
import functools

import jax
import jax.numpy as jnp
from jax import lax
from jax.experimental import pallas as pl
from jax.experimental.pallas import tpu as pltpu
from jax.experimental.pallas import tpu_sc as plsc

N_NODES = 10000
D = 128
N_EDGES = 320000

NC, NS = 2, 16
NW = NC * NS
CHUNK = 128
K = 80
EP = NW * K * CHUNK
NP = 10240
RPT = NP // NS
DW = 16

_mesh = plsc.VectorSubcoreMesh(
    core_axis_name="c", subcore_axis_name="s", num_cores=NC, num_subcores=NS
)


_deg_wrap = functools.partial(
    pl.kernel,
    out_type=jax.ShapeDtypeStruct((NC, NP), jnp.float32),
    mesh=_mesh,
    scratch_types=[
        pltpu.VMEM((K, CHUNK), jnp.int32),
        pltpu.VMEM((CHUNK,), jnp.float32),
        pltpu.VMEM((RPT,), jnp.float32),
        pltpu.VMEM_SHARED((NP,), jnp.float32),
    ],
)


def _deg_body(dst_hbm, out_hbm, idx_v, ones_v, zeros_v, acc):
    c = lax.axis_index("c")
    s = lax.axis_index("s")
    wid = c * NS + s

    def fill_z(r, _):
        zeros_v[pl.ds(r * 16, 16)] = jnp.zeros((16,), jnp.float32)
        return 0

    lax.fori_loop(0, RPT // 16, fill_z, 0)

    def fill_o(r, _):
        ones_v[pl.ds(r * 16, 16)] = jnp.full((16,), 1.0, jnp.float32)
        return 0

    lax.fori_loop(0, CHUNK // 16, fill_o, 0)

    pltpu.sync_copy(zeros_v, acc.at[pl.ds(s * RPT, RPT)])
    pltpu.sync_copy(dst_hbm.at[wid], idx_v)
    plsc.subcore_barrier()

    def body(j, _):
        pltpu.sync_copy(ones_v, acc.at[idx_v.at[j]], add=True)
        return 0

    lax.fori_loop(0, K, body, 0)
    plsc.subcore_barrier()
    pltpu.sync_copy(acc.at[pl.ds(s * RPT, RPT)], out_hbm.at[c, pl.ds(s * RPT, RPT)])


_msg_wrap = functools.partial(
    pl.kernel,
    out_type=jax.ShapeDtypeStruct((NC, NP, D), jnp.float32),
    mesh=_mesh,
    scratch_types=[
        pltpu.VMEM((K, CHUNK), jnp.int32),
        pltpu.VMEM((2, CHUNK), jnp.int32),
        pltpu.VMEM((CHUNK, D), jnp.float32),
        pltpu.VMEM((CHUNK, D), jnp.float32),
        pltpu.VMEM_SHARED((NP, D), jnp.float32),
        pltpu.SemaphoreType.DMA,
        pltpu.SemaphoreType.DMA,
        pltpu.SemaphoreType.DMA,
        pltpu.SemaphoreType.DMA,
    ],
)


def _msg_body(y_hbm, src_hbm, dst_hbm, out_hbm, src_v, dst_v, buf_a, buf_b,
              acc, sem_a, sem_b, sem_da, sem_db):
    c = lax.axis_index("c")
    s = lax.axis_index("s")
    wid = c * NS + s

    def fill(r, _):
        for q in range(D // 16):
            buf_a[r, pl.ds(q * 16, 16)] = jnp.zeros((16,), jnp.float32)
        return 0

    lax.fori_loop(0, CHUNK, fill, 0)

    def zblk(b, _):
        pltpu.sync_copy(buf_a, acc.at[pl.ds(s * RPT + b * CHUNK, CHUNK)])
        return 0

    lax.fori_loop(0, RPT // CHUNK, zblk, 0)
    pltpu.sync_copy(src_hbm.at[wid], src_v)
    plsc.subcore_barrier()

    pltpu.async_copy(dst_hbm.at[wid, 0], dst_v.at[0], sem_da)
    pltpu.async_copy(y_hbm.at[src_v.at[0]], buf_a, sem_a)
    pltpu.async_copy(dst_hbm.at[wid, 1], dst_v.at[1], sem_db)
    pltpu.async_copy(y_hbm.at[src_v.at[1]], buf_b, sem_b)

    def body(t, _):
        j0 = 2 * t
        more = t + 1 < K // 2
        pltpu.make_async_copy(dst_hbm.at[wid, j0], dst_v.at[0], sem_da).wait()
        pltpu.make_async_copy(y_hbm.at[src_v.at[j0]], buf_a, sem_a).wait()
        pltpu.sync_copy(buf_a, acc.at[dst_v.at[0]], add=True)

        @pl.when(more)
        def _():
            pltpu.async_copy(dst_hbm.at[wid, j0 + 2], dst_v.at[0], sem_da)
            pltpu.async_copy(y_hbm.at[src_v.at[j0 + 2]], buf_a, sem_a)

        pltpu.make_async_copy(dst_hbm.at[wid, j0 + 1], dst_v.at[1], sem_db).wait()
        pltpu.make_async_copy(y_hbm.at[src_v.at[j0 + 1]], buf_b, sem_b).wait()
        pltpu.sync_copy(buf_b, acc.at[dst_v.at[1]], add=True)

        @pl.when(more)
        def _():
            pltpu.async_copy(dst_hbm.at[wid, j0 + 3], dst_v.at[1], sem_db)
            pltpu.async_copy(y_hbm.at[src_v.at[j0 + 3]], buf_b, sem_b)

        return 0

    lax.fori_loop(0, K // 2, body, 0)
    plsc.subcore_barrier()
    pltpu.sync_copy(acc.at[pl.ds(s * RPT, RPT)], out_hbm.at[c, pl.ds(s * RPT, RPT)])


_deg_kernel = _deg_wrap(_deg_body)
_msg_kernel = _msg_wrap(_msg_body)


def _tc_pre_body(x_ref, w_ref, deg_ref, xw_ref, y_ref):
    xw = jnp.dot(x_ref[...], w_ref[...], preferred_element_type=jnp.float32)
    deg = deg_ref[0, 0:N_NODES, :] + deg_ref[1, 0:N_NODES, :] + 1.0
    dinv = lax.rsqrt(deg)
    xw_ref[...] = xw
    y_ref[...] = xw * dinv


def _tc_pre(x, W1, degp):
    return pl.pallas_call(
        _tc_pre_body,
        out_shape=(
            jax.ShapeDtypeStruct((N_NODES, D), jnp.float32),
            jax.ShapeDtypeStruct((N_NODES, D), jnp.float32),
        ),
    )(x, W1, degp)


def _tc_mid_body(z_ref, deg_ref, xw1_ref, w2_ref, b1_ref, xw2_ref, y2_ref):
    z = z_ref[0, 0:N_NODES, :] + z_ref[1, 0:N_NODES, :]
    deg = deg_ref[0, 0:N_NODES, :] + deg_ref[1, 0:N_NODES, :] + 1.0
    dinv = lax.rsqrt(deg)
    h = jnp.maximum(z * dinv + xw1_ref[...] * (dinv * dinv) + b1_ref[...], 0.0)
    xw2 = jnp.dot(h, w2_ref[...], preferred_element_type=jnp.float32)
    xw2_ref[...] = xw2
    y2_ref[...] = xw2 * dinv


def _tc_mid(z1p, degp, xw1, W2, b1):
    return pl.pallas_call(
        _tc_mid_body,
        out_shape=(
            jax.ShapeDtypeStruct((N_NODES, D), jnp.float32),
            jax.ShapeDtypeStruct((N_NODES, D), jnp.float32),
        ),
    )(z1p, degp, xw1, W2, b1)


def _tc_post_body(z_ref, deg_ref, xw2_ref, b2_ref, out_ref):
    z = z_ref[0, 0:N_NODES, :] + z_ref[1, 0:N_NODES, :]
    deg = deg_ref[0, 0:N_NODES, :] + deg_ref[1, 0:N_NODES, :] + 1.0
    dinv = lax.rsqrt(deg)
    out_ref[...] = z * dinv + xw2_ref[...] * (dinv * dinv) + b2_ref[...]


def _tc_post(z2p, degp, xw2, b2):
    return pl.pallas_call(
        _tc_post_body,
        out_shape=jax.ShapeDtypeStruct((N_NODES, D), jnp.float32),
    )(z2p, degp, xw2, b2)


def kernel(basic_block, edge_index, W1, b1, W2, b2):
    src = edge_index[0].astype(jnp.int32)
    dst = edge_index[1].astype(jnp.int32)
    pad = EP - N_EDGES
    src_p = jnp.concatenate([src, jnp.zeros((pad,), jnp.int32)]).reshape(NW, K, CHUNK)
    dst_p = jnp.concatenate([dst, jnp.full((pad,), NP - 1, jnp.int32)]).reshape(NW, K, CHUNK)

    degp = _deg_kernel(dst_p).reshape(NC, NP, 1)
    xw1, y1 = _tc_pre(basic_block, W1, degp)
    z1p = _msg_kernel(y1, src_p, dst_p)
    xw2, y2 = _tc_mid(z1p, degp, xw1, W2, b1.reshape(1, D))
    z2p = _msg_kernel(y2, src_p, dst_p)
    return _tc_post(z2p, degp, xw2, b2.reshape(1, D))
